# baseline (speedup 1.0000x reference)
import functools

import jax
import jax.numpy as jnp
from jax.experimental import pallas as pl
from jax.experimental.pallas import tpu as pltpu

_dot = functools.partial(jnp.dot, preferred_element_type=jnp.float32,
                         precision=jax.lax.Precision.HIGHEST)


def _dot_t(a, b):
    return jax.lax.dot_general(a, b, (((1,), (1,)), ((), ())),
                               preferred_element_type=jnp.float32,
                               precision=jax.lax.Precision.HIGHEST)


def _body(x_ref, We_ref, be_ref, Wrel_ref, brel_ref, Wroot_ref, Wdec_ref,
          bdec_ref, ew_ref, es_ref, et_ref, o_ref, w2_scr, b2_scr):
    F = Wrel_ref.shape[0]
    NF = We_ref.shape[1]
    N = NF // F
    E = ew_ref.shape[1]
    f32 = jnp.float32

    @pl.when(pl.program_id(0) == 0)
    def _build():
        nidx = jax.lax.broadcasted_iota(jnp.int32, (N, E), 0)
        src_w = jnp.where(nidx == es_ref[...], ew_ref[...], 0.0)
        tgt_oh = jnp.where(nidx == et_ref[...], 1.0, 0.0)
        A = _dot_t(src_w, tgt_oh)

        Rn = (jax.lax.broadcasted_iota(jnp.int32, (NF, N), 0) // F
              == jax.lax.broadcasted_iota(jnp.int32, (NF, N), 1)).astype(f32)
        Rf = (jax.lax.broadcasted_iota(jnp.int32, (NF, F), 0) % F
              == jax.lax.broadcasted_iota(jnp.int32, (NF, F), 1)).astype(f32)

        def expand(R, W):
            return _dot_t(_dot(R, W), R)

        A_big = expand(Rn, A)
        Wrel_big = expand(Rf, Wrel_ref[...])
        Wroot_big = expand(Rf, Wroot_ref[...])
        blockdiag = (jax.lax.broadcasted_iota(jnp.int32, (NF, NF), 0) // F
                     == jax.lax.broadcasted_iota(jnp.int32, (NF, NF), 1) // F
                     ).astype(f32)
        M = A_big * Wrel_big + blockdiag * Wroot_big

        b80 = _dot_t(brel_ref[...], Rf)
        w2_scr[...] = _dot(M, Wdec_ref[...]).astype(jnp.bfloat16)
        b2_scr[...] = _dot(b80, Wdec_ref[...]) + bdec_ref[...]

    bf16 = jnp.bfloat16
    y = jnp.maximum(
        jnp.dot(x_ref[...], We_ref[...].astype(bf16),
                preferred_element_type=f32) + be_ref[...], 0.0)
    o_ref[...] = jnp.maximum(
        jnp.dot(y.astype(bf16), w2_scr[...],
                preferred_element_type=f32) + b2_scr[...], 0.0)


def kernel(x, W_enc, b_enc, W_rel, b_rel, W_root, W_dec, b_dec,
           edge_weight, edge_src, edge_tgt):
    B, Din = x.shape
    NF = W_enc.shape[1]
    Dout = W_dec.shape[1]
    E = edge_src.shape[0]

    xb = x.astype(jnp.bfloat16)

    es = edge_src.reshape(1, E).astype(jnp.int32)
    et = edge_tgt.reshape(1, E).astype(jnp.int32)
    ew = edge_weight.reshape(1, E).astype(jnp.float32)
    be = b_enc.reshape(1, NF)
    brel = b_rel.reshape(1, -1)
    bdec = b_dec.reshape(1, Dout)

    BLK = min(B, 32768)
    grid = (pl.cdiv(B, BLK),)
    full = lambda s: pl.BlockSpec(s, lambda i: (0, 0))
    return pl.pallas_call(
        _body,
        grid=grid,
        in_specs=[
            pl.BlockSpec((BLK, Din), lambda i: (i, 0)),
            full(W_enc.shape), full(be.shape), full(W_rel.shape),
            full(brel.shape), full(W_root.shape), full(W_dec.shape),
            full(bdec.shape), full(ew.shape), full(es.shape), full(et.shape),
        ],
        out_specs=pl.BlockSpec((BLK, Dout), lambda i: (i, 0)),
        out_shape=jax.ShapeDtypeStruct((B, Dout), jnp.float32),
        scratch_shapes=[pltpu.VMEM((NF, Dout), jnp.bfloat16),
                        pltpu.VMEM((1, Dout), jnp.float32)],
        compiler_params=pltpu.CompilerParams(
            dimension_semantics=("parallel",)),
    )(xb, W_enc, be, W_rel, brel, W_root, W_dec, bdec, ew, es, et)


# speedup vs baseline: 1.0143x; 1.0143x over previous
import functools

import jax
import jax.numpy as jnp
from jax.experimental import pallas as pl
from jax.experimental.pallas import tpu as pltpu

_dot = functools.partial(jnp.dot, preferred_element_type=jnp.float32,
                         precision=jax.lax.Precision.HIGHEST)


def _dot_t(a, b):
    return jax.lax.dot_general(a, b, (((1,), (1,)), ((), ())),
                               preferred_element_type=jnp.float32,
                               precision=jax.lax.Precision.HIGHEST)


def _body(x_ref, We_ref, be_ref, Wrel_ref, brel_ref, Wroot_ref, Wdec_ref,
          bdec_ref, ew_ref, es_ref, et_ref, o_ref, w2_scr, b2_scr):
    F = Wrel_ref.shape[0]
    NF = We_ref.shape[1]
    N = NF // F
    E = ew_ref.shape[1]
    f32 = jnp.float32

    @pl.when(pl.program_id(0) == 0)
    def _build():
        nidx = jax.lax.broadcasted_iota(jnp.int32, (N, E), 0)
        src_w = jnp.where(nidx == es_ref[...], ew_ref[...], 0.0)
        tgt_oh = jnp.where(nidx == et_ref[...], 1.0, 0.0)
        A = _dot_t(src_w, tgt_oh)

        Rn = (jax.lax.broadcasted_iota(jnp.int32, (NF, N), 0) // F
              == jax.lax.broadcasted_iota(jnp.int32, (NF, N), 1)).astype(f32)
        Rf = (jax.lax.broadcasted_iota(jnp.int32, (NF, F), 0) % F
              == jax.lax.broadcasted_iota(jnp.int32, (NF, F), 1)).astype(f32)

        def expand(R, W):
            return _dot_t(_dot(R, W), R)

        A_big = expand(Rn, A)
        Wrel_big = expand(Rf, Wrel_ref[...])
        Wroot_big = expand(Rf, Wroot_ref[...])
        blockdiag = (jax.lax.broadcasted_iota(jnp.int32, (NF, NF), 0) // F
                     == jax.lax.broadcasted_iota(jnp.int32, (NF, NF), 1) // F
                     ).astype(f32)
        M = A_big * Wrel_big + blockdiag * Wroot_big

        b80 = _dot_t(brel_ref[...], Rf)
        w2_scr[...] = _dot(M, Wdec_ref[...]).astype(jnp.bfloat16)
        b2_scr[...] = _dot(b80, Wdec_ref[...]) + bdec_ref[...]

    bf16 = jnp.bfloat16
    y = jnp.maximum(
        jnp.dot(x_ref[...], We_ref[...].astype(bf16),
                preferred_element_type=f32) + be_ref[...], 0.0)
    o_ref[...] = jnp.maximum(
        jnp.dot(y.astype(bf16), w2_scr[...],
                preferred_element_type=f32) + b2_scr[...], 0.0)


def kernel(x, W_enc, b_enc, W_rel, b_rel, W_root, W_dec, b_dec,
           edge_weight, edge_src, edge_tgt):
    B, Din = x.shape
    NF = W_enc.shape[1]
    Dout = W_dec.shape[1]
    E = edge_src.shape[0]

    xb = x.astype(jnp.bfloat16)

    es = edge_src.reshape(1, E).astype(jnp.int32)
    et = edge_tgt.reshape(1, E).astype(jnp.int32)
    ew = edge_weight.reshape(1, E).astype(jnp.float32)
    be = b_enc.reshape(1, NF)
    brel = b_rel.reshape(1, -1)
    bdec = b_dec.reshape(1, Dout)

    BLK = min(B, 20480)
    grid = (pl.cdiv(B, BLK),)
    full = lambda s: pl.BlockSpec(s, lambda i: (0, 0))
    return pl.pallas_call(
        _body,
        grid=grid,
        in_specs=[
            pl.BlockSpec((BLK, Din), lambda i: (i, 0)),
            full(W_enc.shape), full(be.shape), full(W_rel.shape),
            full(brel.shape), full(W_root.shape), full(W_dec.shape),
            full(bdec.shape), full(ew.shape), full(es.shape), full(et.shape),
        ],
        out_specs=pl.BlockSpec((BLK, Dout), lambda i: (i, 0)),
        out_shape=jax.ShapeDtypeStruct((B, Dout), jnp.float32),
        scratch_shapes=[pltpu.VMEM((NF, Dout), jnp.bfloat16),
                        pltpu.VMEM((1, Dout), jnp.float32)],
        compiler_params=pltpu.CompilerParams(
            dimension_semantics=("parallel",)),
    )(xb, W_enc, be, W_rel, brel, W_root, W_dec, bdec, ew, es, et)
